# baseline (speedup 1.0000x reference)
import functools

import jax
import jax.numpy as jnp
from jax.experimental import pallas as pl
from jax.experimental.pallas import tpu as pltpu
from jax.experimental.pallas import tpu_sc as plsc

_DIM = 64
_NE = 1024
_B = 16
_S = 576
_ROWS = _B * _S

_BH = _B // 2
_ROWS_H = _BH * _S

_NC, _NS = 2, 16
_NW = _NC * _NS
_BPW = _ROWS_H // _NW
_NCH = 2
_CH = _BPW // _NCH


def _tc_body(xt_ref, et_ref, idx_ref, acc_ref):
    i = pl.program_id(0)
    xb = xt_ref[0]
    et = et_ref[...]
    s = jax.lax.dot_general(et, xb, (((1,), (0,)), ((), ())),
                            preferred_element_type=jnp.float32)
    x2 = jnp.sum(xb * xb, axis=0, keepdims=True)
    e2 = jnp.sum(et * et, axis=1, keepdims=True)
    sq = (x2 - 2.0 * s) + e2
    m = jnp.min(sq, axis=0, keepdims=True)
    iot = jax.lax.broadcasted_iota(jnp.int32, sq.shape, 0)
    idx = jnp.min(jnp.where(sq == m, iot, _NE), axis=0)
    idx_ref[0, 0, :] = idx
    part = jnp.sum(jnp.maximum(m, 0.0))
    prev = jnp.where(i == 0, 0.0, acc_ref[0, 0])
    acc_ref[0, 0] = prev + part


def _make_tc_call(off):
    return pl.pallas_call(
        _tc_body,
        grid=(_BH,),
        in_specs=[
            pl.BlockSpec((1, _DIM, _S), lambda i: (i + off, 0, 0)),
            pl.BlockSpec((_NE, _DIM), lambda i: (0, 0)),
        ],
        out_specs=[
            pl.BlockSpec((1, 1, _S), lambda i: (i, 0, 0)),
            pl.BlockSpec((1, 1), lambda i: (0, 0), memory_space=pltpu.SMEM),
        ],
        out_shape=[
            jax.ShapeDtypeStruct((_BH, 1, _S), jnp.int32),
            jax.ShapeDtypeStruct((1, 1), jnp.float32),
        ],
    )


_tc_call_a = _make_tc_call(0)
_tc_call_b = _make_tc_call(_BH)


@functools.partial(
    pl.kernel,
    mesh=plsc.VectorSubcoreMesh(core_axis_name="c", subcore_axis_name="s"),
    compiler_params=pltpu.CompilerParams(use_tc_tiling_on_sc=False),
    out_type=jax.ShapeDtypeStruct((_ROWS_H, _DIM), jnp.float32),
    scratch_types=[
        pltpu.VMEM((_NCH, _CH), jnp.int32),
        pltpu.VMEM((_BPW, _DIM), jnp.float32),
        pltpu.SemaphoreType.DMA,
    ],
)
def _sc_gather(table_hbm, idx_hbm, out_hbm, idx_v, rows_v, sem):
    wid = jax.lax.axis_index("s") * _NC + jax.lax.axis_index("c")
    base = wid * _BPW
    pltpu.sync_copy(idx_hbm.at[wid], idx_v)
    copies = [
        pltpu.async_copy(
            table_hbm.at[idx_v.at[j]],
            rows_v.at[pl.ds(j * _CH, _CH)],
            sem,
        )
        for j in range(_NCH)
    ]
    for c in copies:
        c.wait()
    pltpu.sync_copy(rows_v, out_hbm.at[pl.ds(base, _BPW)])


def kernel(input, embed):
    xt = jnp.transpose(input, (0, 2, 1))
    table = embed.T
    idx_a, acc_a = _tc_call_a(xt, table)
    quant_a = _sc_gather(table, idx_a.reshape(_NW, _NCH, _CH))
    idx_b, acc_b = _tc_call_b(xt, table)
    quant_b = _sc_gather(table, idx_b.reshape(_NW, _NCH, _CH))
    diff = (acc_a[0, 0] + acc_b[0, 0]) * (1.0 / (_ROWS * _DIM))
    quantize = jnp.concatenate([quant_a, quant_b], axis=0)
    idx = jnp.concatenate([idx_a.reshape(_BH, _S), idx_b.reshape(_BH, _S)], 0)
    return quantize.reshape(input.shape), diff, idx


# speedup vs baseline: 1.0107x; 1.0107x over previous
import functools

import jax
import jax.numpy as jnp
from jax.experimental import pallas as pl
from jax.experimental.pallas import tpu as pltpu
from jax.experimental.pallas import tpu_sc as plsc

_DIM = 64
_NE = 1024
_B = 16
_S = 576
_ROWS = _B * _S

_NC, _NS = 2, 16
_NW = _NC * _NS
_BPW = _ROWS // _NW
_NCH = 3
_CH = _BPW // _NCH


def _tc_body(xt_ref, et_ref, idx_ref, acc_ref):
    i = pl.program_id(0)
    xb = xt_ref[0]
    et = et_ref[...]
    s = jax.lax.dot_general(et, xb, (((1,), (0,)), ((), ())),
                            preferred_element_type=jnp.float32)
    x2 = jnp.sum(xb * xb, axis=0, keepdims=True)
    e2 = jnp.sum(et * et, axis=1, keepdims=True)
    sq = (x2 - 2.0 * s) + e2
    m = jnp.min(sq, axis=0, keepdims=True)
    iot = jax.lax.broadcasted_iota(jnp.int32, sq.shape, 0)
    idx = jnp.min(jnp.where(sq == m, iot, _NE), axis=0)
    idx_ref[0, 0, :] = idx
    part = jnp.sum(jnp.maximum(m, 0.0))
    prev = jnp.where(i == 0, 0.0, acc_ref[0, 0])
    tot = prev + part
    acc_ref[0, 0] = jnp.where(i == _B - 1, tot * (1.0 / (_ROWS * _DIM)), tot)


_tc_call = pl.pallas_call(
    _tc_body,
    grid=(_B,),
    in_specs=[
        pl.BlockSpec((1, _DIM, _S), lambda i: (i, 0, 0)),
        pl.BlockSpec((_NE, _DIM), lambda i: (0, 0)),
    ],
    out_specs=[
        pl.BlockSpec((1, 1, _S), lambda i: (i, 0, 0)),
        pl.BlockSpec((1, 1), lambda i: (0, 0), memory_space=pltpu.SMEM),
    ],
    out_shape=[
        jax.ShapeDtypeStruct((_B, 1, _S), jnp.int32),
        jax.ShapeDtypeStruct((1, 1), jnp.float32),
    ],
)


@functools.partial(
    pl.kernel,
    mesh=plsc.VectorSubcoreMesh(core_axis_name="c", subcore_axis_name="s"),
    compiler_params=pltpu.CompilerParams(use_tc_tiling_on_sc=False),
    out_type=jax.ShapeDtypeStruct((_ROWS, _DIM), jnp.float32),
    scratch_types=[
        pltpu.VMEM((_NCH, _CH), jnp.int32),
        pltpu.VMEM((_BPW, _DIM), jnp.float32),
        pltpu.SemaphoreType.DMA,
        pltpu.SemaphoreType.DMA,
    ],
)
def _sc_gather(table_hbm, idx_hbm, out_hbm, idx_v, rows_v, gsem, wsem):
    wid = jax.lax.axis_index("s") * _NC + jax.lax.axis_index("c")
    base = wid * _BPW
    pltpu.sync_copy(idx_hbm.at[wid], idx_v)
    gathers = [
        pltpu.async_copy(
            table_hbm.at[idx_v.at[j]],
            rows_v.at[pl.ds(j * _CH, _CH)],
            gsem,
        )
        for j in range(_NCH)
    ]
    writes = []
    for j in range(_NCH):
        gathers[j].wait()
        writes.append(
            pltpu.async_copy(
                rows_v.at[pl.ds(j * _CH, _CH)],
                out_hbm.at[pl.ds(base + j * _CH, _CH)],
                wsem,
            )
        )
    for w in writes:
        w.wait()


def kernel(input, embed):
    xt = jnp.transpose(input, (0, 2, 1))
    table = embed.T
    idx3, acc = _tc_call(xt, table)
    quantize = _sc_gather(table, idx3.reshape(_NW, _NCH, _CH))
    diff = acc[0, 0]
    return quantize.reshape(input.shape), diff, idx3.reshape(_B, _S)


# speedup vs baseline: 1.0851x; 1.0736x over previous
import functools

import jax
import jax.numpy as jnp
from jax.experimental import pallas as pl
from jax.experimental.pallas import tpu as pltpu
from jax.experimental.pallas import tpu_sc as plsc

_DIM = 64
_NE = 1024
_B = 16
_S = 576
_ROWS = _B * _S

_NC, _NS = 2, 16
_NW = _NC * _NS
_BPW = _ROWS // _NW
_NCH = 3
_CH = _BPW // _NCH


_BB = 4
_NSTEP = _B // _BB


def _tc_body(xt_ref, et_ref, idx_ref, acc_ref):
    i = pl.program_id(0)
    et = et_ref[...]
    e2 = jnp.sum(et * et, axis=1, keepdims=True)
    prev = jnp.where(i == 0, 0.0, acc_ref[0, 0])
    for k in range(_BB):
        xb = xt_ref[k]
        s = jax.lax.dot_general(et, xb, (((1,), (0,)), ((), ())),
                                preferred_element_type=jnp.float32)
        x2 = jnp.sum(xb * xb, axis=0, keepdims=True)
        sq = (x2 - 2.0 * s) + e2
        m = jnp.min(sq, axis=0, keepdims=True)
        iot = jax.lax.broadcasted_iota(jnp.int32, sq.shape, 0)
        idx = jnp.min(jnp.where(sq == m, iot, _NE), axis=0)
        idx_ref[k, 0, :] = idx
        prev = prev + jnp.sum(jnp.maximum(m, 0.0))
    acc_ref[0, 0] = jnp.where(
        i == _NSTEP - 1, prev * (1.0 / (_ROWS * _DIM)), prev)


_tc_call = pl.pallas_call(
    _tc_body,
    grid=(_NSTEP,),
    in_specs=[
        pl.BlockSpec((_BB, _DIM, _S), lambda i: (i, 0, 0)),
        pl.BlockSpec((_NE, _DIM), lambda i: (0, 0)),
    ],
    out_specs=[
        pl.BlockSpec((_BB, 1, _S), lambda i: (i, 0, 0)),
        pl.BlockSpec((1, 1), lambda i: (0, 0), memory_space=pltpu.SMEM),
    ],
    out_shape=[
        jax.ShapeDtypeStruct((_B, 1, _S), jnp.int32),
        jax.ShapeDtypeStruct((1, 1), jnp.float32),
    ],
)


@functools.partial(
    pl.kernel,
    mesh=plsc.VectorSubcoreMesh(core_axis_name="c", subcore_axis_name="s"),
    compiler_params=pltpu.CompilerParams(use_tc_tiling_on_sc=False),
    out_type=jax.ShapeDtypeStruct((_ROWS, _DIM), jnp.float32),
    scratch_types=[
        pltpu.VMEM((_NCH, _CH), jnp.int32),
        pltpu.VMEM((_BPW, _DIM), jnp.float32),
        pltpu.SemaphoreType.DMA,
    ],
)
def _sc_gather(table_hbm, idx_hbm, out_hbm, idx_v, rows_v, sem):
    wid = jax.lax.axis_index("s") * _NC + jax.lax.axis_index("c")
    base = wid * _BPW
    pltpu.sync_copy(idx_hbm.at[wid], idx_v)
    copies = [
        pltpu.async_copy(
            table_hbm.at[idx_v.at[j]],
            rows_v.at[pl.ds(j * _CH, _CH)],
            sem,
        )
        for j in range(_NCH)
    ]
    for c in copies:
        c.wait()
    pltpu.sync_copy(rows_v, out_hbm.at[pl.ds(base, _BPW)])


def kernel(input, embed):
    xt = jnp.transpose(input, (0, 2, 1))
    table = embed.T
    idx3, acc = _tc_call(xt, table)
    quantize = _sc_gather(table, idx3.reshape(_NW, _NCH, _CH))
    diff = acc[0, 0]
    return quantize.reshape(input.shape), diff, idx3.reshape(_B, _S)
